# baseline (speedup 1.0000x reference)
import functools

import jax
import jax.numpy as jnp
from jax import lax
from jax.experimental import pallas as pl
from jax.experimental.pallas import tpu as pltpu
from jax.experimental.pallas import tpu_sc as plsc

NC = 2
NS = 16
NW = NC * NS
L = 16

N = 200
D = 64
B = 4096
BLK = 1024
CBLK = 16
CHUNK = CBLK * BLK
BPW = 1600
NSTEP = BPW // CBLK
NSLOT = 4
TOT = N * D * B


def _compute(buf, pbv, g):
    @pl.loop(0, CBLK)
    def _(j):
        u = (g * CBLK + j) // 32
        for dl in range(8):
            splat = pbv[pl.ds((u * 8 + dl) * L, L)]
            for u8 in range(8):
                plsc.addupdate(
                    buf.at[pl.ds(j * BLK + dl * 128 + u8 * L, L)], splat)


def _sc_body(x_hbm, pb_hbm, out_hbm, tb0, tb1, pb_v, sp,
             s1a, s1b, s1c, s1d, s5a, s5b, s5c, s5d, s2a, s2b, s4a, s4b):
    c = lax.axis_index("c")
    s = lax.axis_index("s")
    w = s * NC + c
    blk0 = w * BPW
    pltpu.sync_copy(pb_hbm.at[pl.ds(w * (BPW // 32) * 8 * L,
                                    (BPW // 32) * 8 * L)], pb_v)

    tbufs = (tb0, tb1)
    s1 = (s1a, s1b, s1c, s1d)
    s5 = (s5a, s5b, s5c, s5d)
    s2 = (s2a, s2b)
    s4 = (s4a, s4b)

    def hbm_in(g, slot):
        return pltpu.make_async_copy(
            x_hbm.at[pl.ds((blk0 + g * CBLK) * BLK, CHUNK)],
            sp.at[s, slot], s1[slot])

    def to_tile(g, slot, tb):
        return pltpu.make_async_copy(sp.at[s, slot], tbufs[tb], s2[tb])

    def from_tile(g, slot, tb):
        return pltpu.make_async_copy(tbufs[tb], sp.at[s, slot], s4[tb])

    def hbm_out(g, slot):
        return pltpu.make_async_copy(
            sp.at[s, slot],
            out_hbm.at[pl.ds((blk0 + g * CBLK) * BLK, CHUNK)], s5[slot])

    hbm_in(0, 0).start()
    hbm_in(1, 1).start()
    hbm_in(0, 0).wait()
    to_tile(0, 0, 0).start()

    @pl.loop(0, NSTEP)
    def _(g):
        for m in range(NSLOT):
            @pl.when(g % NSLOT == m)
            def _(g=g, m=m):
                tb = m % 2
                pm1, pp1, pm2, pp2 = ((m - 1) % NSLOT, (m + 1) % NSLOT,
                                      (m - 2) % NSLOT, (m + 2) % NSLOT)

                @pl.when(g >= 1)
                def _():
                    from_tile(g - 1, pm1, 1 - tb).wait()
                    hbm_out(g - 1, pm1).start()

                @pl.when(g >= 2)
                def _():
                    hbm_out(g - 2, pm2).wait()

                @pl.when(g + 2 < NSTEP)
                def _():
                    hbm_in(g + 2, pm2).start()

                @pl.when(g + 1 < NSTEP)
                def _():
                    hbm_in(g + 1, pp1).wait()
                    to_tile(g + 1, pp1, 1 - tb).start()

                to_tile(g, m, tb).wait()
                _compute(tbufs[tb], pb_v, g)
                from_tile(g, m, tb).start()

    for m in range(NSLOT):
        @pl.when((NSTEP - 1) % NSLOT == m)
        def _(m=m):
            tb = m % 2
            from_tile(NSTEP - 1, m, tb).wait()
            hbm_out(NSTEP - 1, m).start()
            hbm_out(NSTEP - 2, (m - 1) % NSLOT).wait()
            hbm_out(NSTEP - 1, m).wait()


_sc_call = functools.partial(
    pl.kernel,
    out_type=jax.ShapeDtypeStruct((TOT,), jnp.float32),
    mesh=plsc.VectorSubcoreMesh(core_axis_name="c", subcore_axis_name="s"),
    scratch_types=[
        pltpu.VMEM((CHUNK,), jnp.float32),
        pltpu.VMEM((CHUNK,), jnp.float32),
        pltpu.VMEM(((BPW // 32) * 8 * L,), jnp.float32),
        pltpu.VMEM_SHARED((NS, NSLOT, CHUNK), jnp.float32),
        pltpu.SemaphoreType.DMA,
        pltpu.SemaphoreType.DMA,
        pltpu.SemaphoreType.DMA,
        pltpu.SemaphoreType.DMA,
        pltpu.SemaphoreType.DMA,
        pltpu.SemaphoreType.DMA,
        pltpu.SemaphoreType.DMA,
        pltpu.SemaphoreType.DMA,
        pltpu.SemaphoreType.DMA,
        pltpu.SemaphoreType.DMA,
        pltpu.SemaphoreType.DMA,
        pltpu.SemaphoreType.DMA,
    ],
)(_sc_body)


def kernel(x, pos_table):
    t = jnp.transpose(x, (1, 2, 0))
    r = t.reshape(N, 8, 8, 32, 128)
    x0 = jnp.transpose(r, (0, 1, 3, 2, 4)).reshape(TOT)

    posf = pos_table[:N].reshape(N * D)
    pb16 = jnp.repeat(posf, L)

    out0 = _sc_call(x0, pb16)

    o = out0.reshape(N, 8, 32, 8, 128)
    o = jnp.transpose(o, (0, 1, 3, 2, 4)).reshape(N, D, B)
    return jnp.transpose(o, (2, 0, 1))


# speedup vs baseline: 1.4209x; 1.4209x over previous
import functools

import jax
import jax.numpy as jnp
from jax import lax
from jax.experimental import pallas as pl
from jax.experimental.pallas import tpu as pltpu
from jax.experimental.pallas import tpu_sc as plsc

NC = 2
NS = 16
NW = NC * NS
L = 16

R = 12800
B = 4096
RPW = R // NW
RC = 8
NSTEP = RPW // RC
COLV = B // L


def _compute(buf, pbv, g):
    for r in range(RC):
        splat = pbv[pl.ds((g * RC + r) * L, L)]

        @pl.loop(0, COLV, unroll=8)
        def _(i, splat=splat, r=r):
            plsc.addupdate(buf.at[r, pl.ds(i * L, L)], splat)


def _sc_body(x_hbm, pb_hbm, out_hbm, buf0, buf1, pb_v, si0, si1, so0, so1):
    c = lax.axis_index("c")
    s = lax.axis_index("s")
    w = s * NC + c
    row0 = w * RPW
    pltpu.sync_copy(pb_hbm.at[pl.ds(row0 * L, RPW * L)], pb_v)

    bufs = (buf0, buf1)
    sin = (si0, si1)
    sout = (so0, so1)

    def in_cp(g, b):
        return pltpu.make_async_copy(
            x_hbm.at[pl.ds(row0 + g * RC, RC)], bufs[b], sin[b])

    def out_cp(g, b):
        return pltpu.make_async_copy(
            bufs[b], out_hbm.at[pl.ds(row0 + g * RC, RC)], sout[b])

    in_cp(0, 0).start()

    @pl.loop(0, NSTEP, step=2)
    def _(g0):
        for b in range(2):
            g = g0 + b

            @pl.when(g >= 1)
            def _(g=g, b=b):
                out_cp(g - 1, 1 - b).wait()

            @pl.when(g + 1 < NSTEP)
            def _(g=g, b=b):
                in_cp(g + 1, 1 - b).start()

            in_cp(g, b).wait()
            _compute(bufs[b], pb_v, g)
            out_cp(g, b).start()

    out_cp(NSTEP - 1, 1).wait()


_sc_call = functools.partial(
    pl.kernel,
    out_type=jax.ShapeDtypeStruct((R, B), jnp.float32),
    mesh=plsc.VectorSubcoreMesh(core_axis_name="c", subcore_axis_name="s"),
    scratch_types=[
        pltpu.VMEM((RC, B), jnp.float32),
        pltpu.VMEM((RC, B), jnp.float32),
        pltpu.VMEM((RPW * L,), jnp.float32),
        pltpu.SemaphoreType.DMA,
        pltpu.SemaphoreType.DMA,
        pltpu.SemaphoreType.DMA,
        pltpu.SemaphoreType.DMA,
    ],
)(_sc_body)


def kernel(x, pos_table):
    Bx, n, d = x.shape
    x2 = jnp.transpose(x, (1, 2, 0)).reshape(R, B)
    posf = pos_table[:n].reshape(R)
    pb16 = jnp.repeat(posf, L)
    out2 = _sc_call(x2, pb16)
    return jnp.transpose(out2.reshape(n, d, Bx), (2, 0, 1))
